# baseline (speedup 1.0000x reference)
import functools

import jax
import jax.numpy as jnp
from jax import lax
from jax.experimental import pallas as pl
from jax.experimental.pallas import tpu as pltpu
from jax.experimental.pallas import tpu_sc as plsc

N = 10000
E = 320000
DIN = 128
H = 512
C = 7
NTR = 65536

NC = 2
NS = 16
NW = NC * NS


_EB = 80
_K = 4
_NIT = 32
_EPW = _NIT * _K * _EB
EPAD = NW * _EPW
NP = 10240
_RPT = NP // NS
_ZR = 32


def _sc_segsum_builder(nchunks):
    mesh = plsc.VectorSubcoreMesh(core_axis_name="c", subcore_axis_name="s")

    @functools.partial(
        pl.kernel,
        out_type=jax.ShapeDtypeStruct((NC, nchunks, NP, DIN), jnp.float32),
        mesh=mesh,
        scratch_types=[pltpu.VMEM((_EB,), jnp.int32) for _ in range(2 * _K)]
        + [pltpu.VMEM((_EB, DIN), jnp.float32) for _ in range(_K)] + [
            pltpu.VMEM((_ZR, DIN), jnp.float32),
            pltpu.VMEM_SHARED((NP, DIN), jnp.float32),
        ] + [pltpu.SemaphoreType.DMA for _ in range(2 * _K)],
    )
    def segsum(tab4, src_hbm, dst_hbm, out_hbm, *rest):
        srcv = rest[:_K]
        dstv = rest[_K:2 * _K]
        rows = rest[2 * _K:3 * _K]
        zbuf = rest[3 * _K]
        acc = rest[3 * _K + 1]
        gsem = rest[3 * _K + 2:4 * _K + 2]
        ssem = rest[4 * _K + 2:]
        cid = lax.axis_index("c")
        sid = lax.axis_index("s")
        wid = sid * NC + cid

        def zrow(i, carry):
            z = jnp.zeros((16,), jnp.float32)
            for j in range(DIN // 16):
                zbuf[i, pl.ds(j * 16, 16)] = z
            return carry

        lax.fori_loop(0, _ZR, zrow, 0)

        for ci in range(nchunks):
            def zcopy(i, carry):
                pltpu.sync_copy(zbuf, acc.at[pl.ds(sid * _RPT + i * _ZR, _ZR)])
                return carry

            lax.fori_loop(0, _RPT // _ZR, zcopy, 0)
            plsc.subcore_barrier()

            tab = tab4.at[ci]

            def body(i, carry):
                base = wid * _EPW + i * (_K * _EB)
                for k in range(_K):
                    pltpu.sync_copy(
                        src_hbm.at[pl.ds(base + k * _EB, _EB)], srcv[k])
                    pltpu.sync_copy(
                        dst_hbm.at[pl.ds(base + k * _EB, _EB)], dstv[k])
                gs = []
                for k in range(_K):
                    gs.append(pltpu.async_copy(
                        tab.at[srcv[k]], rows[k], gsem[k]))
                ss = []
                for k in range(_K):
                    gs[k].wait()
                    ss.append(pltpu.async_copy(
                        rows[k], acc.at[dstv[k]], ssem[k], add=True))
                for k in range(_K):
                    ss[k].wait()
                return carry

            lax.fori_loop(0, _NIT, body, 0)
            plsc.subcore_barrier()
            pltpu.sync_copy(
                acc.at[pl.ds(sid * _RPT, _RPT)],
                out_hbm.at[cid, ci, pl.ds(sid * _RPT, _RPT)],
            )
            if ci + 1 < nchunks:
                plsc.subcore_barrier()

    return segsum


_segsum1 = _sc_segsum_builder(1)
_segsum4 = _sc_segsum_builder(4)


_PPW = NTR // NW
_PB = 64
_PNB = _PPW // _PB

_head_mesh = plsc.VectorSubcoreMesh(core_axis_name="c", subcore_axis_name="s")


@functools.partial(
    pl.kernel,
    out_type=jax.ShapeDtypeStruct((NTR, H), jnp.float32),
    mesh=_head_mesh,
    scratch_types=[
        pltpu.VMEM((_PB,), jnp.int32),
        pltpu.VMEM((_PB,), jnp.int32),
        pltpu.VMEM((_PB,), jnp.int32),
        pltpu.VMEM((_PB, H), jnp.float32),
        pltpu.VMEM((_PB, H), jnp.float32),
        pltpu.SemaphoreType.DMA,
    ],
)
def _sc_head(h3_hbm, src_hbm, dst_hbm, teid_hbm, out_hbm,
             teidv, av, bv, arows, brows, sem):
    cid = lax.axis_index("c")
    sid = lax.axis_index("s")
    wid = sid * NC + cid

    def body(b, carry):
        base = wid * _PPW + b * _PB
        pltpu.sync_copy(teid_hbm.at[pl.ds(base, _PB)], teidv)
        pltpu.async_copy(src_hbm.at[teidv], av, sem).wait()
        pltpu.async_copy(dst_hbm.at[teidv], bv, sem).wait()
        pltpu.async_copy(h3_hbm.at[av], arows, sem).wait()
        pltpu.async_copy(h3_hbm.at[bv], brows, sem).wait()

        def prod(i, c2):
            for j in range(H // 16):
                sl = pl.ds(j * 16, 16)
                arows[i, sl] = arows[i, sl] * brows[i, sl]
            return c2

        lax.fori_loop(0, _PB, prod, 0)
        pltpu.sync_copy(arows, out_hbm.at[pl.ds(base, _PB)])
        return carry

    lax.fori_loop(0, _PNB, body, 0)


_BR = 2000


def _mlp1_body(eps_ref, x_ref, p0_ref, p1_ref, w1_ref, b1_ref, w2_ref,
               b2_ref, s1_ref, t1_ref, out_ref):
    xin = (1.0 + eps_ref[0]) * x_ref[...] + p0_ref[0, 0] + p1_ref[0, 0]
    h = jnp.dot(xin, w1_ref[...], preferred_element_type=jnp.float32)
    h = jnp.maximum(h + b1_ref[...], 0.0)
    h = jnp.dot(h, w2_ref[...], preferred_element_type=jnp.float32)
    h = jnp.maximum(h + b2_ref[...], 0.0)
    h = h * s1_ref[...] + t1_ref[...]
    for c in range(4):
        out_ref[c] = h[:, c * DIN:(c + 1) * DIN]


def _tc_mlp1(x, parts, eps1, W1, b1, W2, b2, s1, t1):
    grid = (N // _BR,)
    return pl.pallas_call(
        _mlp1_body,
        grid=grid,
        in_specs=[
            pl.BlockSpec(memory_space=pltpu.SMEM),
            pl.BlockSpec((_BR, DIN), lambda i: (i, 0)),
            pl.BlockSpec((1, 1, _BR, DIN), lambda i: (0, 0, i, 0)),
            pl.BlockSpec((1, 1, _BR, DIN), lambda i: (1, 0, i, 0)),
            pl.BlockSpec((DIN, H), lambda i: (0, 0)),
            pl.BlockSpec((1, H), lambda i: (0, 0)),
            pl.BlockSpec((H, H), lambda i: (0, 0)),
            pl.BlockSpec((1, H), lambda i: (0, 0)),
            pl.BlockSpec((1, H), lambda i: (0, 0)),
            pl.BlockSpec((1, H), lambda i: (0, 0)),
        ],
        out_specs=pl.BlockSpec((4, _BR, DIN), lambda i: (0, i, 0)),
        out_shape=jax.ShapeDtypeStruct((4, N, DIN), jnp.float32),
    )(eps1, x, parts, parts, W1, b1, W2, b2, s1, t1)


def _mlp2_body(eps_ref, h4_ref, q0_ref, q1_ref, w3_ref, b3_ref, s2_ref,
               t2_ref, wl_ref, bl_ref, out_ref):
    h = jnp.concatenate([h4_ref[c] for c in range(4)], axis=1)
    agg = jnp.concatenate(
        [q0_ref[0, c] + q1_ref[0, c] for c in range(4)], axis=1)
    hin = (1.0 + eps_ref[0]) * h + agg
    h2 = jnp.dot(hin, w3_ref[...], preferred_element_type=jnp.float32)
    h2 = jnp.maximum(h2 + b3_ref[...], 0.0)
    h2 = h2 * s2_ref[...] + t2_ref[...]
    h3 = jnp.dot(h2, wl_ref[...], preferred_element_type=jnp.float32)
    out_ref[...] = jnp.maximum(h3 + bl_ref[...], 0.0)


def _tc_mlp2(h4, parts2, eps2, W3, b3, s2, t2, Wl, bl):
    grid = (N // _BR,)
    return pl.pallas_call(
        _mlp2_body,
        grid=grid,
        in_specs=[
            pl.BlockSpec(memory_space=pltpu.SMEM),
            pl.BlockSpec((4, _BR, DIN), lambda i: (0, i, 0)),
            pl.BlockSpec((1, 4, _BR, DIN), lambda i: (0, 0, i, 0)),
            pl.BlockSpec((1, 4, _BR, DIN), lambda i: (1, 0, i, 0)),
            pl.BlockSpec((H, H), lambda i: (0, 0)),
            pl.BlockSpec((1, H), lambda i: (0, 0)),
            pl.BlockSpec((1, H), lambda i: (0, 0)),
            pl.BlockSpec((1, H), lambda i: (0, 0)),
            pl.BlockSpec((H, H), lambda i: (0, 0)),
            pl.BlockSpec((1, H), lambda i: (0, 0)),
        ],
        out_specs=pl.BlockSpec((_BR, H), lambda i: (i, 0)),
        out_shape=jax.ShapeDtypeStruct((N, H), jnp.float32),
    )(eps2, h4, parts2, parts2, W3, b3, s2, t2, Wl, bl)


_BH = 4096


def _headmm_body(p_ref, wf_ref, bf_ref, out_ref):
    o = jnp.dot(p_ref[...], wf_ref[...], preferred_element_type=jnp.float32)
    out_ref[...] = o + bf_ref[...]


def _tc_headmm(p, Wf, bf):
    grid = (NTR // _BH,)
    return pl.pallas_call(
        _headmm_body,
        grid=grid,
        in_specs=[
            pl.BlockSpec((_BH, H), lambda i: (i, 0)),
            pl.BlockSpec((H, C), lambda i: (0, 0)),
            pl.BlockSpec((1, C), lambda i: (0, 0)),
        ],
        out_specs=pl.BlockSpec((_BH, C), lambda i: (i, 0)),
        out_shape=jax.ShapeDtypeStruct((NTR, C), jnp.float32),
    )(p, Wf, bf)


_BN_RS = float(1.0 / (1.0 + 1e-5) ** 0.5)


def kernel(x, edge_index, train_edge_id, eps1, W1, b1, W2, b2, g1, bb1,
           eps2, W3, b3, g2, bb2, Wl, bl, Wf, bf):
    src = edge_index[0]
    dst = edge_index[1]
    eps1s = jnp.reshape(eps1, (1,))
    eps2s = jnp.reshape(eps2, (1,))
    s1 = jnp.reshape(g1 * _BN_RS, (1, H))
    t1 = jnp.reshape(bb1, (1, H))
    s2 = jnp.reshape(g2 * _BN_RS, (1, H))
    t2 = jnp.reshape(bb2, (1, H))
    b1r = jnp.reshape(b1, (1, H))
    b2r = jnp.reshape(b2, (1, H))
    b3r = jnp.reshape(b3, (1, H))
    blr = jnp.reshape(bl, (1, H))
    bfr = jnp.reshape(bf, (1, C))

    pad = EPAD - E
    src_pad = jnp.arange(pad, dtype=jnp.int32) % N
    src3 = jnp.concatenate([src, src_pad])
    dst_pad = N + jnp.arange(pad, dtype=jnp.int32) % (NP - N)
    dst3 = jnp.concatenate([dst, dst_pad])
    parts = _segsum1(jnp.reshape(x, (1, N, DIN)), src3, dst3)
    h4 = _tc_mlp1(x, parts, eps1s, W1, b1r, W2, b2r, s1, t1)
    parts2 = _segsum4(h4, src3, dst3)
    h3 = _tc_mlp2(h4, parts2, eps2s, W3, b3r, s2, t2, Wl, blr)
    p = _sc_head(h3, src, dst, train_edge_id)
    return _tc_headmm(p, Wf, bfr)


# speedup vs baseline: 1.0385x; 1.0385x over previous
import functools

import jax
import jax.numpy as jnp
from jax import lax
from jax.experimental import pallas as pl
from jax.experimental.pallas import tpu as pltpu
from jax.experimental.pallas import tpu_sc as plsc

N = 10000
E = 320000
DIN = 128
H = 512
C = 7
NTR = 65536

NC = 2
NS = 16
NW = NC * NS


_EB = 128
_K = 1
_NIT = 80
_EPW = _NIT * _K * _EB
EPAD = NW * _EPW
NP = 10240
_RPT = NP // NS
_ZR = 32


def _sc_segsum_builder(nchunks):
    mesh = plsc.VectorSubcoreMesh(core_axis_name="c", subcore_axis_name="s")

    @functools.partial(
        pl.kernel,
        out_type=jax.ShapeDtypeStruct((NC, nchunks, NP, DIN), jnp.float32),
        mesh=mesh,
        scratch_types=[pltpu.VMEM((_EB,), jnp.int32) for _ in range(2 * _K)]
        + [pltpu.VMEM((_EB, DIN), jnp.float32) for _ in range(_K)] + [
            pltpu.VMEM((_ZR, DIN), jnp.float32),
            pltpu.VMEM_SHARED((NP, DIN), jnp.float32),
        ] + [pltpu.SemaphoreType.DMA for _ in range(2 * _K)],
    )
    def segsum(tab4, src_hbm, dst_hbm, out_hbm, *rest):
        srcv = rest[:_K]
        dstv = rest[_K:2 * _K]
        rows = rest[2 * _K:3 * _K]
        zbuf = rest[3 * _K]
        acc = rest[3 * _K + 1]
        gsem = rest[3 * _K + 2:4 * _K + 2]
        ssem = rest[4 * _K + 2:]
        cid = lax.axis_index("c")
        sid = lax.axis_index("s")
        wid = sid * NC + cid

        def zrow(i, carry):
            z = jnp.zeros((16,), jnp.float32)
            for j in range(DIN // 16):
                zbuf[i, pl.ds(j * 16, 16)] = z
            return carry

        lax.fori_loop(0, _ZR, zrow, 0)

        for ci in range(nchunks):
            def zcopy(i, carry):
                pltpu.sync_copy(zbuf, acc.at[pl.ds(sid * _RPT + i * _ZR, _ZR)])
                return carry

            lax.fori_loop(0, _RPT // _ZR, zcopy, 0)
            plsc.subcore_barrier()

            tab = tab4.at[ci]

            def body(i, carry):
                base = wid * _EPW + i * (_K * _EB)
                for k in range(_K):
                    pltpu.sync_copy(
                        src_hbm.at[pl.ds(base + k * _EB, _EB)], srcv[k])
                    pltpu.sync_copy(
                        dst_hbm.at[pl.ds(base + k * _EB, _EB)], dstv[k])
                gs = []
                for k in range(_K):
                    gs.append(pltpu.async_copy(
                        tab.at[srcv[k]], rows[k], gsem[k]))
                ss = []
                for k in range(_K):
                    gs[k].wait()
                    ss.append(pltpu.async_copy(
                        rows[k], acc.at[dstv[k]], ssem[k], add=True))
                for k in range(_K):
                    ss[k].wait()
                return carry

            lax.fori_loop(0, _NIT, body, 0)
            plsc.subcore_barrier()
            pltpu.sync_copy(
                acc.at[pl.ds(sid * _RPT, _RPT)],
                out_hbm.at[cid, ci, pl.ds(sid * _RPT, _RPT)],
            )
            if ci + 1 < nchunks:
                plsc.subcore_barrier()

    return segsum


_segsum1 = _sc_segsum_builder(1)
_segsum4 = _sc_segsum_builder(4)


_PPW = NTR // NW
_PB = 64
_PNB = _PPW // _PB

_head_mesh = plsc.VectorSubcoreMesh(core_axis_name="c", subcore_axis_name="s")


@functools.partial(
    pl.kernel,
    out_type=jax.ShapeDtypeStruct((NTR, H), jnp.float32),
    mesh=_head_mesh,
    scratch_types=[
        pltpu.VMEM((_PB,), jnp.int32),
        pltpu.VMEM((_PB,), jnp.int32),
        pltpu.VMEM((_PB,), jnp.int32),
        pltpu.VMEM((_PB, H), jnp.float32),
        pltpu.VMEM((_PB, H), jnp.float32),
        pltpu.SemaphoreType.DMA,
    ],
)
def _sc_head(h3_hbm, src_hbm, dst_hbm, teid_hbm, out_hbm,
             teidv, av, bv, arows, brows, sem):
    cid = lax.axis_index("c")
    sid = lax.axis_index("s")
    wid = sid * NC + cid

    def body(b, carry):
        base = wid * _PPW + b * _PB
        pltpu.sync_copy(teid_hbm.at[pl.ds(base, _PB)], teidv)
        pltpu.async_copy(src_hbm.at[teidv], av, sem).wait()
        pltpu.async_copy(dst_hbm.at[teidv], bv, sem).wait()
        pltpu.async_copy(h3_hbm.at[av], arows, sem).wait()
        pltpu.async_copy(h3_hbm.at[bv], brows, sem).wait()

        def prod(i, c2):
            for j in range(H // 16):
                sl = pl.ds(j * 16, 16)
                arows[i, sl] = arows[i, sl] * brows[i, sl]
            return c2

        lax.fori_loop(0, _PB, prod, 0)
        pltpu.sync_copy(arows, out_hbm.at[pl.ds(base, _PB)])
        return carry

    lax.fori_loop(0, _PNB, body, 0)


_BR = 2000


def _mlp1_body(eps_ref, x_ref, p0_ref, p1_ref, w1_ref, b1_ref, w2_ref,
               b2_ref, s1_ref, t1_ref, out_ref):
    xin = (1.0 + eps_ref[0]) * x_ref[...] + p0_ref[0, 0] + p1_ref[0, 0]
    h = jnp.dot(xin, w1_ref[...], preferred_element_type=jnp.float32)
    h = jnp.maximum(h + b1_ref[...], 0.0)
    h = jnp.dot(h, w2_ref[...], preferred_element_type=jnp.float32)
    h = jnp.maximum(h + b2_ref[...], 0.0)
    h = h * s1_ref[...] + t1_ref[...]
    for c in range(4):
        out_ref[c] = h[:, c * DIN:(c + 1) * DIN]


def _tc_mlp1(x, parts, eps1, W1, b1, W2, b2, s1, t1):
    grid = (N // _BR,)
    return pl.pallas_call(
        _mlp1_body,
        grid=grid,
        in_specs=[
            pl.BlockSpec(memory_space=pltpu.SMEM),
            pl.BlockSpec((_BR, DIN), lambda i: (i, 0)),
            pl.BlockSpec((1, 1, _BR, DIN), lambda i: (0, 0, i, 0)),
            pl.BlockSpec((1, 1, _BR, DIN), lambda i: (1, 0, i, 0)),
            pl.BlockSpec((DIN, H), lambda i: (0, 0)),
            pl.BlockSpec((1, H), lambda i: (0, 0)),
            pl.BlockSpec((H, H), lambda i: (0, 0)),
            pl.BlockSpec((1, H), lambda i: (0, 0)),
            pl.BlockSpec((1, H), lambda i: (0, 0)),
            pl.BlockSpec((1, H), lambda i: (0, 0)),
        ],
        out_specs=pl.BlockSpec((4, _BR, DIN), lambda i: (0, i, 0)),
        out_shape=jax.ShapeDtypeStruct((4, N, DIN), jnp.float32),
    )(eps1, x, parts, parts, W1, b1, W2, b2, s1, t1)


def _mlp2_body(eps_ref, h4_ref, q0_ref, q1_ref, w3_ref, b3_ref, s2_ref,
               t2_ref, wl_ref, bl_ref, out_ref):
    h = jnp.concatenate([h4_ref[c] for c in range(4)], axis=1)
    agg = jnp.concatenate(
        [q0_ref[0, c] + q1_ref[0, c] for c in range(4)], axis=1)
    hin = (1.0 + eps_ref[0]) * h + agg
    h2 = jnp.dot(hin, w3_ref[...], preferred_element_type=jnp.float32)
    h2 = jnp.maximum(h2 + b3_ref[...], 0.0)
    h2 = h2 * s2_ref[...] + t2_ref[...]
    h3 = jnp.dot(h2, wl_ref[...], preferred_element_type=jnp.float32)
    out_ref[...] = jnp.maximum(h3 + bl_ref[...], 0.0)


def _tc_mlp2(h4, parts2, eps2, W3, b3, s2, t2, Wl, bl):
    grid = (N // _BR,)
    return pl.pallas_call(
        _mlp2_body,
        grid=grid,
        in_specs=[
            pl.BlockSpec(memory_space=pltpu.SMEM),
            pl.BlockSpec((4, _BR, DIN), lambda i: (0, i, 0)),
            pl.BlockSpec((1, 4, _BR, DIN), lambda i: (0, 0, i, 0)),
            pl.BlockSpec((1, 4, _BR, DIN), lambda i: (1, 0, i, 0)),
            pl.BlockSpec((H, H), lambda i: (0, 0)),
            pl.BlockSpec((1, H), lambda i: (0, 0)),
            pl.BlockSpec((1, H), lambda i: (0, 0)),
            pl.BlockSpec((1, H), lambda i: (0, 0)),
            pl.BlockSpec((H, H), lambda i: (0, 0)),
            pl.BlockSpec((1, H), lambda i: (0, 0)),
        ],
        out_specs=pl.BlockSpec((_BR, H), lambda i: (i, 0)),
        out_shape=jax.ShapeDtypeStruct((N, H), jnp.float32),
    )(eps2, h4, parts2, parts2, W3, b3, s2, t2, Wl, bl)


_BH = 4096


def _headmm_body(p_ref, wf_ref, bf_ref, out_ref):
    o = jnp.dot(p_ref[...], wf_ref[...], preferred_element_type=jnp.float32)
    out_ref[...] = o + bf_ref[...]


def _tc_headmm(p, Wf, bf):
    grid = (NTR // _BH,)
    return pl.pallas_call(
        _headmm_body,
        grid=grid,
        in_specs=[
            pl.BlockSpec((_BH, H), lambda i: (i, 0)),
            pl.BlockSpec((H, C), lambda i: (0, 0)),
            pl.BlockSpec((1, C), lambda i: (0, 0)),
        ],
        out_specs=pl.BlockSpec((_BH, C), lambda i: (i, 0)),
        out_shape=jax.ShapeDtypeStruct((NTR, C), jnp.float32),
    )(p, Wf, bf)


_BN_RS = float(1.0 / (1.0 + 1e-5) ** 0.5)


def kernel(x, edge_index, train_edge_id, eps1, W1, b1, W2, b2, g1, bb1,
           eps2, W3, b3, g2, bb2, Wl, bl, Wf, bf):
    src = edge_index[0]
    dst = edge_index[1]
    eps1s = jnp.reshape(eps1, (1,))
    eps2s = jnp.reshape(eps2, (1,))
    s1 = jnp.reshape(g1 * _BN_RS, (1, H))
    t1 = jnp.reshape(bb1, (1, H))
    s2 = jnp.reshape(g2 * _BN_RS, (1, H))
    t2 = jnp.reshape(bb2, (1, H))
    b1r = jnp.reshape(b1, (1, H))
    b2r = jnp.reshape(b2, (1, H))
    b3r = jnp.reshape(b3, (1, H))
    blr = jnp.reshape(bl, (1, H))
    bfr = jnp.reshape(bf, (1, C))

    pad = EPAD - E
    src_pad = jnp.arange(pad, dtype=jnp.int32) % N
    src3 = jnp.concatenate([src, src_pad])
    dst_pad = N + jnp.arange(pad, dtype=jnp.int32) % (NP - N)
    dst3 = jnp.concatenate([dst, dst_pad])
    parts = _segsum1(jnp.reshape(x, (1, N, DIN)), src3, dst3)
    h4 = _tc_mlp1(x, parts, eps1s, W1, b1r, W2, b2r, s1, t1)
    parts2 = _segsum4(h4, src3, dst3)
    h3 = _tc_mlp2(h4, parts2, eps2s, W3, b3r, s2, t2, Wl, blr)
    p = _sc_head(h3, src, dst, train_edge_id)
    return _tc_headmm(p, Wf, bfr)


# speedup vs baseline: 1.2859x; 1.2382x over previous
import functools

import jax
import jax.numpy as jnp
from jax import lax
from jax.experimental import pallas as pl
from jax.experimental.pallas import tpu as pltpu
from jax.experimental.pallas import tpu_sc as plsc

N = 10000
E = 320000
DIN = 128
H = 512
C = 7
NTR = 65536

NC = 2
NS = 16
NW = NC * NS


_EB = 80
_K = 4
_NIT = 32
_EPW = _NIT * _K * _EB
EPAD = NW * _EPW
NP = 10240
_RPT = NP // NS
_ZR = 32


def _sc_segsum_builder(nchunks):
    mesh = plsc.VectorSubcoreMesh(core_axis_name="c", subcore_axis_name="s")

    @functools.partial(
        pl.kernel,
        out_type=jax.ShapeDtypeStruct((NC, nchunks, NP, DIN), jnp.float32),
        mesh=mesh,
        scratch_types=[
            pltpu.VMEM((_K, _EB), jnp.int32),
            pltpu.VMEM((_K, _EB), jnp.int32),
        ] + [pltpu.VMEM((_EB, DIN), jnp.float32) for _ in range(_K)] + [
            pltpu.VMEM((_ZR, DIN), jnp.float32),
            pltpu.VMEM_SHARED((NP, DIN), jnp.float32),
        ] + [pltpu.SemaphoreType.DMA for _ in range(2 * _K)],
    )
    def segsum(tab4, src_hbm, dst_hbm, out_hbm, *rest):
        srcv = rest[0]
        dstv = rest[1]
        rows = rest[2:2 + _K]
        zbuf = rest[2 + _K]
        acc = rest[3 + _K]
        gsem = rest[4 + _K:4 + 2 * _K]
        ssem = rest[4 + 2 * _K:]
        cid = lax.axis_index("c")
        sid = lax.axis_index("s")
        wid = sid * NC + cid

        def zrow(i, carry):
            z = jnp.zeros((16,), jnp.float32)
            for j in range(DIN // 16):
                zbuf[i, pl.ds(j * 16, 16)] = z
            return carry

        lax.fori_loop(0, _ZR, zrow, 0)

        for ci in range(nchunks):
            def zcopy(i, carry):
                pltpu.sync_copy(zbuf, acc.at[pl.ds(sid * _RPT + i * _ZR, _ZR)])
                return carry

            lax.fori_loop(0, _RPT // _ZR, zcopy, 0)
            plsc.subcore_barrier()

            tab = tab4.at[ci]

            def body(i, carry):
                pltpu.sync_copy(src_hbm.at[wid, i], srcv)
                pltpu.sync_copy(dst_hbm.at[wid, i], dstv)
                gs = []
                for k in range(_K):
                    gs.append(pltpu.async_copy(
                        tab.at[srcv.at[k]], rows[k], gsem[k]))
                ss = []
                for k in range(_K):
                    gs[k].wait()
                    ss.append(pltpu.async_copy(
                        rows[k], acc.at[dstv.at[k]], ssem[k], add=True))
                for k in range(_K):
                    ss[k].wait()
                return carry

            lax.fori_loop(0, _NIT, body, 0)
            plsc.subcore_barrier()
            pltpu.sync_copy(
                acc.at[pl.ds(sid * _RPT, _RPT)],
                out_hbm.at[cid, ci, pl.ds(sid * _RPT, _RPT)],
            )
            if ci + 1 < nchunks:
                plsc.subcore_barrier()

    return segsum


_segsum1 = _sc_segsum_builder(1)
_segsum4 = _sc_segsum_builder(4)


_PPW = NTR // NW
_PB = 64
_PNB = _PPW // _PB

_head_mesh = plsc.VectorSubcoreMesh(core_axis_name="c", subcore_axis_name="s")


@functools.partial(
    pl.kernel,
    out_type=jax.ShapeDtypeStruct((NTR, H), jnp.float32),
    mesh=_head_mesh,
    scratch_types=[
        pltpu.VMEM((_PB,), jnp.int32),
        pltpu.VMEM((_PB,), jnp.int32),
        pltpu.VMEM((_PB,), jnp.int32),
        pltpu.VMEM((_PB, H), jnp.float32),
        pltpu.VMEM((_PB, H), jnp.float32),
        pltpu.SemaphoreType.DMA,
    ],
)
def _sc_head(h3_hbm, src_hbm, dst_hbm, teid_hbm, out_hbm,
             teidv, av, bv, arows, brows, sem):
    cid = lax.axis_index("c")
    sid = lax.axis_index("s")
    wid = sid * NC + cid

    def body(b, carry):
        base = wid * _PPW + b * _PB
        pltpu.sync_copy(teid_hbm.at[pl.ds(base, _PB)], teidv)
        pltpu.async_copy(src_hbm.at[teidv], av, sem).wait()
        pltpu.async_copy(dst_hbm.at[teidv], bv, sem).wait()
        pltpu.async_copy(h3_hbm.at[av], arows, sem).wait()
        pltpu.async_copy(h3_hbm.at[bv], brows, sem).wait()

        def prod(i, c2):
            for j in range(H // 16):
                sl = pl.ds(j * 16, 16)
                arows[i, sl] = arows[i, sl] * brows[i, sl]
            return c2

        lax.fori_loop(0, _PB, prod, 0)
        pltpu.sync_copy(arows, out_hbm.at[pl.ds(base, _PB)])
        return carry

    lax.fori_loop(0, _PNB, body, 0)


_BR = 2000


def _mlp1_body(eps_ref, x_ref, p0_ref, p1_ref, w1_ref, b1_ref, w2_ref,
               b2_ref, s1_ref, t1_ref, out_ref):
    xin = (1.0 + eps_ref[0]) * x_ref[...] + p0_ref[0, 0] + p1_ref[0, 0]
    h = jnp.dot(xin, w1_ref[...], preferred_element_type=jnp.float32)
    h = jnp.maximum(h + b1_ref[...], 0.0)
    h = jnp.dot(h, w2_ref[...], preferred_element_type=jnp.float32)
    h = jnp.maximum(h + b2_ref[...], 0.0)
    h = h * s1_ref[...] + t1_ref[...]
    for c in range(4):
        out_ref[c] = h[:, c * DIN:(c + 1) * DIN]


def _tc_mlp1(x, parts, eps1, W1, b1, W2, b2, s1, t1):
    grid = (N // _BR,)
    return pl.pallas_call(
        _mlp1_body,
        grid=grid,
        in_specs=[
            pl.BlockSpec(memory_space=pltpu.SMEM),
            pl.BlockSpec((_BR, DIN), lambda i: (i, 0)),
            pl.BlockSpec((1, 1, _BR, DIN), lambda i: (0, 0, i, 0)),
            pl.BlockSpec((1, 1, _BR, DIN), lambda i: (1, 0, i, 0)),
            pl.BlockSpec((DIN, H), lambda i: (0, 0)),
            pl.BlockSpec((1, H), lambda i: (0, 0)),
            pl.BlockSpec((H, H), lambda i: (0, 0)),
            pl.BlockSpec((1, H), lambda i: (0, 0)),
            pl.BlockSpec((1, H), lambda i: (0, 0)),
            pl.BlockSpec((1, H), lambda i: (0, 0)),
        ],
        out_specs=pl.BlockSpec((4, _BR, DIN), lambda i: (0, i, 0)),
        out_shape=jax.ShapeDtypeStruct((4, N, DIN), jnp.float32),
    )(eps1, x, parts, parts, W1, b1, W2, b2, s1, t1)


def _mlp2_body(eps_ref, h4_ref, q0_ref, q1_ref, w3_ref, b3_ref, s2_ref,
               t2_ref, wl_ref, bl_ref, out_ref):
    h = jnp.concatenate([h4_ref[c] for c in range(4)], axis=1)
    agg = jnp.concatenate(
        [q0_ref[0, c] + q1_ref[0, c] for c in range(4)], axis=1)
    hin = (1.0 + eps_ref[0]) * h + agg
    h2 = jnp.dot(hin, w3_ref[...], preferred_element_type=jnp.float32)
    h2 = jnp.maximum(h2 + b3_ref[...], 0.0)
    h2 = h2 * s2_ref[...] + t2_ref[...]
    h3 = jnp.dot(h2, wl_ref[...], preferred_element_type=jnp.float32)
    out_ref[...] = jnp.maximum(h3 + bl_ref[...], 0.0)


def _tc_mlp2(h4, parts2, eps2, W3, b3, s2, t2, Wl, bl):
    grid = (N // _BR,)
    return pl.pallas_call(
        _mlp2_body,
        grid=grid,
        in_specs=[
            pl.BlockSpec(memory_space=pltpu.SMEM),
            pl.BlockSpec((4, _BR, DIN), lambda i: (0, i, 0)),
            pl.BlockSpec((1, 4, _BR, DIN), lambda i: (0, 0, i, 0)),
            pl.BlockSpec((1, 4, _BR, DIN), lambda i: (1, 0, i, 0)),
            pl.BlockSpec((H, H), lambda i: (0, 0)),
            pl.BlockSpec((1, H), lambda i: (0, 0)),
            pl.BlockSpec((1, H), lambda i: (0, 0)),
            pl.BlockSpec((1, H), lambda i: (0, 0)),
            pl.BlockSpec((H, H), lambda i: (0, 0)),
            pl.BlockSpec((1, H), lambda i: (0, 0)),
        ],
        out_specs=pl.BlockSpec((_BR, H), lambda i: (i, 0)),
        out_shape=jax.ShapeDtypeStruct((N, H), jnp.float32),
    )(eps2, h4, parts2, parts2, W3, b3, s2, t2, Wl, bl)


_BH = 4096


def _headmm_body(p_ref, wf_ref, bf_ref, out_ref):
    o = jnp.dot(p_ref[...], wf_ref[...], preferred_element_type=jnp.float32)
    out_ref[...] = o + bf_ref[...]


def _tc_headmm(p, Wf, bf):
    grid = (NTR // _BH,)
    return pl.pallas_call(
        _headmm_body,
        grid=grid,
        in_specs=[
            pl.BlockSpec((_BH, H), lambda i: (i, 0)),
            pl.BlockSpec((H, C), lambda i: (0, 0)),
            pl.BlockSpec((1, C), lambda i: (0, 0)),
        ],
        out_specs=pl.BlockSpec((_BH, C), lambda i: (i, 0)),
        out_shape=jax.ShapeDtypeStruct((NTR, C), jnp.float32),
    )(p, Wf, bf)


_BN_RS = float(1.0 / (1.0 + 1e-5) ** 0.5)


def kernel(x, edge_index, train_edge_id, eps1, W1, b1, W2, b2, g1, bb1,
           eps2, W3, b3, g2, bb2, Wl, bl, Wf, bf):
    src = edge_index[0]
    dst = edge_index[1]
    eps1s = jnp.reshape(eps1, (1,))
    eps2s = jnp.reshape(eps2, (1,))
    s1 = jnp.reshape(g1 * _BN_RS, (1, H))
    t1 = jnp.reshape(bb1, (1, H))
    s2 = jnp.reshape(g2 * _BN_RS, (1, H))
    t2 = jnp.reshape(bb2, (1, H))
    b1r = jnp.reshape(b1, (1, H))
    b2r = jnp.reshape(b2, (1, H))
    b3r = jnp.reshape(b3, (1, H))
    blr = jnp.reshape(bl, (1, H))
    bfr = jnp.reshape(bf, (1, C))

    pad = EPAD - E
    src_pad = jnp.arange(pad, dtype=jnp.int32) % N
    src3 = jnp.reshape(jnp.concatenate([src, src_pad]),
                       (NW, _NIT, _K, _EB))
    dst_pad = N + jnp.arange(pad, dtype=jnp.int32) % (NP - N)
    dst3 = jnp.reshape(jnp.concatenate([dst, dst_pad]),
                       (NW, _NIT, _K, _EB))
    parts = _segsum1(jnp.reshape(x, (1, N, DIN)), src3, dst3)
    h4 = _tc_mlp1(x, parts, eps1s, W1, b1r, W2, b2r, s1, t1)
    parts2 = _segsum4(h4, src3, dst3)
    h3 = _tc_mlp2(h4, parts2, eps2s, W3, b3r, s2, t2, Wl, blr)
    p = _sc_head(h3, src, dst, train_edge_id)
    return _tc_headmm(p, Wf, bfr)


# speedup vs baseline: 1.4963x; 1.1636x over previous
import functools

import jax
import jax.numpy as jnp
from jax import lax
from jax.experimental import pallas as pl
from jax.experimental.pallas import tpu as pltpu
from jax.experimental.pallas import tpu_sc as plsc

N = 10000
E = 320000
DIN = 128
H = 512
C = 7
NTR = 65536

NC = 2
NS = 16
NW = NC * NS


_EB = 80
_K = 4
_NIT = 32
_EPW = _NIT * _K * _EB
EPAD = NW * _EPW
NP = 10240
_RPT = NP // NS
_ZR = 32


def _sc_segsum_builder(nchunks):
    mesh = plsc.VectorSubcoreMesh(core_axis_name="c", subcore_axis_name="s")

    @functools.partial(
        pl.kernel,
        out_type=jax.ShapeDtypeStruct((NC, nchunks, NP, DIN), jnp.float32),
        mesh=mesh,
        scratch_types=[
            pltpu.VMEM((_K, _EB), jnp.int32),
            pltpu.VMEM((_K, _EB), jnp.int32),
            pltpu.VMEM((_K, _EB), jnp.int32),
            pltpu.VMEM((_K, _EB), jnp.int32),
        ] + [pltpu.VMEM((_EB, DIN), jnp.float32) for _ in range(_K)] + [
            pltpu.VMEM((_ZR, DIN), jnp.float32),
            pltpu.VMEM_SHARED((NP, DIN), jnp.float32),
        ] + [pltpu.SemaphoreType.DMA for _ in range(2 * _K + 2)],
    )
    def segsum(tab4, src_hbm, dst_hbm, out_hbm, *rest):
        srcv = rest[0:2]
        dstv = rest[2:4]
        rows = rest[4:4 + _K]
        zbuf = rest[4 + _K]
        acc = rest[5 + _K]
        gsem = rest[6 + _K:6 + 2 * _K]
        ssem = rest[6 + 2 * _K:6 + 3 * _K]
        isem = rest[6 + 3 * _K:]
        cid = lax.axis_index("c")
        sid = lax.axis_index("s")
        wid = sid * NC + cid

        def idx_wait(p):
            pltpu.make_async_copy(src_hbm.at[wid, 0], srcv[p], isem[p]).wait()
            pltpu.make_async_copy(dst_hbm.at[wid, 0], dstv[p], isem[p]).wait()

        def zrow(i, carry):
            z = jnp.zeros((16,), jnp.float32)
            for j in range(DIN // 16):
                zbuf[i, pl.ds(j * 16, 16)] = z
            return carry

        lax.fori_loop(0, _ZR, zrow, 0)

        for ci in range(nchunks):
            def zcopy(i, carry):
                pltpu.sync_copy(zbuf, acc.at[pl.ds(sid * _RPT + i * _ZR, _ZR)])
                return carry

            lax.fori_loop(0, _RPT // _ZR, zcopy, 0)
            plsc.subcore_barrier()

            tab = tab4.at[ci]

            for p in range(2):
                pltpu.async_copy(src_hbm.at[wid, p], srcv[p], isem[p])
                pltpu.async_copy(dst_hbm.at[wid, p], dstv[p], isem[p])

            def body(i, carry):
                for p in range(2):
                    it = i * 2 + p
                    idx_wait(p)
                    gs = []
                    for k in range(_K):
                        gs.append(pltpu.async_copy(
                            tab.at[srcv[p].at[k]], rows[k], gsem[k]))
                    ss = []
                    for k in range(_K):
                        gs[k].wait()
                        ss.append(pltpu.async_copy(
                            rows[k], acc.at[dstv[p].at[k]], ssem[k],
                            add=True))
                    nxt = jnp.minimum(it + 2, _NIT - 1)
                    pltpu.async_copy(src_hbm.at[wid, nxt], srcv[p], isem[p])
                    for k in range(_K):
                        ss[k].wait()
                    pltpu.async_copy(dst_hbm.at[wid, nxt], dstv[p], isem[p])
                return carry

            lax.fori_loop(0, _NIT // 2, body, 0)
            idx_wait(0)
            idx_wait(1)
            plsc.subcore_barrier()
            pltpu.sync_copy(
                acc.at[pl.ds(sid * _RPT, _RPT)],
                out_hbm.at[cid, ci, pl.ds(sid * _RPT, _RPT)],
            )
            if ci + 1 < nchunks:
                plsc.subcore_barrier()

    return segsum


_segsum1 = _sc_segsum_builder(1)
_segsum4 = _sc_segsum_builder(4)


_PPW = NTR // NW
_PB = 64
_PNB = _PPW // _PB

_head_mesh = plsc.VectorSubcoreMesh(core_axis_name="c", subcore_axis_name="s")


@functools.partial(
    pl.kernel,
    out_type=jax.ShapeDtypeStruct((NTR, H), jnp.float32),
    mesh=_head_mesh,
    scratch_types=[
        pltpu.VMEM((_PB,), jnp.int32),
        pltpu.VMEM((_PB,), jnp.int32),
        pltpu.VMEM((_PB,), jnp.int32),
        pltpu.VMEM((_PB, H), jnp.float32),
        pltpu.VMEM((_PB, H), jnp.float32),
        pltpu.SemaphoreType.DMA,
    ],
)
def _sc_head(h3_hbm, src_hbm, dst_hbm, teid_hbm, out_hbm,
             teidv, av, bv, arows, brows, sem):
    cid = lax.axis_index("c")
    sid = lax.axis_index("s")
    wid = sid * NC + cid

    def body(b, carry):
        base = wid * _PPW + b * _PB
        pltpu.sync_copy(teid_hbm.at[pl.ds(base, _PB)], teidv)
        pltpu.async_copy(src_hbm.at[teidv], av, sem).wait()
        pltpu.async_copy(dst_hbm.at[teidv], bv, sem).wait()
        pltpu.async_copy(h3_hbm.at[av], arows, sem).wait()
        pltpu.async_copy(h3_hbm.at[bv], brows, sem).wait()

        def prod(i, c2):
            for j in range(H // 16):
                sl = pl.ds(j * 16, 16)
                arows[i, sl] = arows[i, sl] * brows[i, sl]
            return c2

        lax.fori_loop(0, _PB, prod, 0)
        pltpu.sync_copy(arows, out_hbm.at[pl.ds(base, _PB)])
        return carry

    lax.fori_loop(0, _PNB, body, 0)


_BR = 2000


def _mlp1_body(eps_ref, x_ref, p0_ref, p1_ref, w1_ref, b1_ref, w2_ref,
               b2_ref, s1_ref, t1_ref, out_ref):
    xin = (1.0 + eps_ref[0]) * x_ref[...] + p0_ref[0, 0] + p1_ref[0, 0]
    h = jnp.dot(xin, w1_ref[...], preferred_element_type=jnp.float32)
    h = jnp.maximum(h + b1_ref[...], 0.0)
    h = jnp.dot(h, w2_ref[...], preferred_element_type=jnp.float32)
    h = jnp.maximum(h + b2_ref[...], 0.0)
    h = h * s1_ref[...] + t1_ref[...]
    for c in range(4):
        out_ref[c] = h[:, c * DIN:(c + 1) * DIN]


def _tc_mlp1(x, parts, eps1, W1, b1, W2, b2, s1, t1):
    grid = (N // _BR,)
    return pl.pallas_call(
        _mlp1_body,
        grid=grid,
        in_specs=[
            pl.BlockSpec(memory_space=pltpu.SMEM),
            pl.BlockSpec((_BR, DIN), lambda i: (i, 0)),
            pl.BlockSpec((1, 1, _BR, DIN), lambda i: (0, 0, i, 0)),
            pl.BlockSpec((1, 1, _BR, DIN), lambda i: (1, 0, i, 0)),
            pl.BlockSpec((DIN, H), lambda i: (0, 0)),
            pl.BlockSpec((1, H), lambda i: (0, 0)),
            pl.BlockSpec((H, H), lambda i: (0, 0)),
            pl.BlockSpec((1, H), lambda i: (0, 0)),
            pl.BlockSpec((1, H), lambda i: (0, 0)),
            pl.BlockSpec((1, H), lambda i: (0, 0)),
        ],
        out_specs=pl.BlockSpec((4, _BR, DIN), lambda i: (0, i, 0)),
        out_shape=jax.ShapeDtypeStruct((4, N, DIN), jnp.float32),
    )(eps1, x, parts, parts, W1, b1, W2, b2, s1, t1)


def _mlp2_body(eps_ref, h4_ref, q0_ref, q1_ref, w3_ref, b3_ref, s2_ref,
               t2_ref, wl_ref, bl_ref, out_ref):
    h = jnp.concatenate([h4_ref[c] for c in range(4)], axis=1)
    agg = jnp.concatenate(
        [q0_ref[0, c] + q1_ref[0, c] for c in range(4)], axis=1)
    hin = (1.0 + eps_ref[0]) * h + agg
    h2 = jnp.dot(hin, w3_ref[...], preferred_element_type=jnp.float32)
    h2 = jnp.maximum(h2 + b3_ref[...], 0.0)
    h2 = h2 * s2_ref[...] + t2_ref[...]
    h3 = jnp.dot(h2, wl_ref[...], preferred_element_type=jnp.float32)
    out_ref[...] = jnp.maximum(h3 + bl_ref[...], 0.0)


def _tc_mlp2(h4, parts2, eps2, W3, b3, s2, t2, Wl, bl):
    grid = (N // _BR,)
    return pl.pallas_call(
        _mlp2_body,
        grid=grid,
        in_specs=[
            pl.BlockSpec(memory_space=pltpu.SMEM),
            pl.BlockSpec((4, _BR, DIN), lambda i: (0, i, 0)),
            pl.BlockSpec((1, 4, _BR, DIN), lambda i: (0, 0, i, 0)),
            pl.BlockSpec((1, 4, _BR, DIN), lambda i: (1, 0, i, 0)),
            pl.BlockSpec((H, H), lambda i: (0, 0)),
            pl.BlockSpec((1, H), lambda i: (0, 0)),
            pl.BlockSpec((1, H), lambda i: (0, 0)),
            pl.BlockSpec((1, H), lambda i: (0, 0)),
            pl.BlockSpec((H, H), lambda i: (0, 0)),
            pl.BlockSpec((1, H), lambda i: (0, 0)),
        ],
        out_specs=pl.BlockSpec((_BR, H), lambda i: (i, 0)),
        out_shape=jax.ShapeDtypeStruct((N, H), jnp.float32),
    )(eps2, h4, parts2, parts2, W3, b3, s2, t2, Wl, bl)


_BH = 4096


def _headmm_body(p_ref, wf_ref, bf_ref, out_ref):
    o = jnp.dot(p_ref[...], wf_ref[...], preferred_element_type=jnp.float32)
    out_ref[...] = o + bf_ref[...]


def _tc_headmm(p, Wf, bf):
    grid = (NTR // _BH,)
    return pl.pallas_call(
        _headmm_body,
        grid=grid,
        in_specs=[
            pl.BlockSpec((_BH, H), lambda i: (i, 0)),
            pl.BlockSpec((H, C), lambda i: (0, 0)),
            pl.BlockSpec((1, C), lambda i: (0, 0)),
        ],
        out_specs=pl.BlockSpec((_BH, C), lambda i: (i, 0)),
        out_shape=jax.ShapeDtypeStruct((NTR, C), jnp.float32),
    )(p, Wf, bf)


_BN_RS = float(1.0 / (1.0 + 1e-5) ** 0.5)


def kernel(x, edge_index, train_edge_id, eps1, W1, b1, W2, b2, g1, bb1,
           eps2, W3, b3, g2, bb2, Wl, bl, Wf, bf):
    src = edge_index[0]
    dst = edge_index[1]
    eps1s = jnp.reshape(eps1, (1,))
    eps2s = jnp.reshape(eps2, (1,))
    s1 = jnp.reshape(g1 * _BN_RS, (1, H))
    t1 = jnp.reshape(bb1, (1, H))
    s2 = jnp.reshape(g2 * _BN_RS, (1, H))
    t2 = jnp.reshape(bb2, (1, H))
    b1r = jnp.reshape(b1, (1, H))
    b2r = jnp.reshape(b2, (1, H))
    b3r = jnp.reshape(b3, (1, H))
    blr = jnp.reshape(bl, (1, H))
    bfr = jnp.reshape(bf, (1, C))

    pad = EPAD - E
    src_pad = jnp.arange(pad, dtype=jnp.int32) % N
    src3 = jnp.reshape(jnp.concatenate([src, src_pad]),
                       (NW, _NIT, _K, _EB))
    dst_pad = N + jnp.arange(pad, dtype=jnp.int32) % (NP - N)
    dst3 = jnp.reshape(jnp.concatenate([dst, dst_pad]),
                       (NW, _NIT, _K, _EB))
    parts = _segsum1(jnp.reshape(x, (1, N, DIN)), src3, dst3)
    h4 = _tc_mlp1(x, parts, eps1s, W1, b1r, W2, b2r, s1, t1)
    parts2 = _segsum4(h4, src3, dst3)
    h3 = _tc_mlp2(h4, parts2, eps2s, W3, b3r, s2, t2, Wl, blr)
    p = _sc_head(h3, src, dst, train_edge_id)
    return _tc_headmm(p, Wf, bfr)


# speedup vs baseline: 1.6160x; 1.0800x over previous
import functools

import jax
import jax.numpy as jnp
from jax import lax
from jax.experimental import pallas as pl
from jax.experimental.pallas import tpu as pltpu
from jax.experimental.pallas import tpu_sc as plsc

N = 10000
E = 320000
DIN = 128
H = 512
C = 7
NTR = 65536

NC = 2
NS = 16
NW = NC * NS


_EB = 80
_K = 4
_NIT = 32
_EPW = _NIT * _K * _EB
EPAD = NW * _EPW
NP = 10240
_RPT = NP // NS
_ZR = 32


def _sc_segsum_builder(nchunks):
    mesh = plsc.VectorSubcoreMesh(core_axis_name="c", subcore_axis_name="s")

    @functools.partial(
        pl.kernel,
        out_type=jax.ShapeDtypeStruct((NC, nchunks, NP, DIN), jnp.float32),
        mesh=mesh,
        scratch_types=[
            pltpu.VMEM((_K, _EB), jnp.int32),
            pltpu.VMEM((_K, _EB), jnp.int32),
            pltpu.VMEM((_K, _EB), jnp.int32),
            pltpu.VMEM((_K, _EB), jnp.int32),
        ] + [pltpu.VMEM((_EB, DIN), jnp.float32) for _ in range(_K)] + [
            pltpu.VMEM((_ZR, DIN), jnp.float32),
            pltpu.VMEM_SHARED((NP, DIN), jnp.float32),
        ] + [pltpu.SemaphoreType.DMA for _ in range(2 * _K + 2)],
    )
    def segsum(tab4, src_hbm, dst_hbm, out_hbm, *rest):
        srcv = rest[0:2]
        dstv = rest[2:4]
        rows = rest[4:4 + _K]
        zbuf = rest[4 + _K]
        acc = rest[5 + _K]
        gsem = rest[6 + _K:6 + 2 * _K]
        ssem = rest[6 + 2 * _K:6 + 3 * _K]
        isem = rest[6 + 3 * _K:]
        cid = lax.axis_index("c")
        sid = lax.axis_index("s")
        wid = sid * NC + cid

        def idx_wait(p):
            pltpu.make_async_copy(src_hbm.at[wid, 0], srcv[p], isem[p]).wait()
            pltpu.make_async_copy(dst_hbm.at[wid, 0], dstv[p], isem[p]).wait()

        def zrow(i, carry):
            z = jnp.zeros((16,), jnp.float32)
            for j in range(DIN // 16):
                zbuf[i, pl.ds(j * 16, 16)] = z
            return carry

        lax.fori_loop(0, _ZR, zrow, 0)

        for ci in range(nchunks):
            def zcopy(i, carry):
                pltpu.sync_copy(zbuf, acc.at[pl.ds(sid * _RPT + i * _ZR, _ZR)])
                return carry

            lax.fori_loop(0, _RPT // _ZR, zcopy, 0)
            plsc.subcore_barrier()

            tab = tab4.at[ci]

            for p in range(2):
                pltpu.async_copy(src_hbm.at[wid, p], srcv[p], isem[p])
                pltpu.async_copy(dst_hbm.at[wid, p], dstv[p], isem[p])

            def body(i, carry):
                for p in range(2):
                    it = i * 2 + p
                    idx_wait(p)
                    gs = []
                    for k in range(_K):
                        gs.append(pltpu.async_copy(
                            tab.at[srcv[p].at[k]], rows[k], gsem[k]))
                    ss = []
                    for k in range(_K):
                        gs[k].wait()
                        ss.append(pltpu.async_copy(
                            rows[k], acc.at[dstv[p].at[k]], ssem[k],
                            add=True))
                    nxt = jnp.minimum(it + 2, _NIT - 1)
                    pltpu.async_copy(src_hbm.at[wid, nxt], srcv[p], isem[p])
                    for k in range(_K):
                        ss[k].wait()
                    pltpu.async_copy(dst_hbm.at[wid, nxt], dstv[p], isem[p])
                return carry

            lax.fori_loop(0, _NIT // 2, body, 0)
            idx_wait(0)
            idx_wait(1)
            plsc.subcore_barrier()
            pltpu.sync_copy(
                acc.at[pl.ds(sid * _RPT, _RPT)],
                out_hbm.at[cid, ci, pl.ds(sid * _RPT, _RPT)],
            )
            if ci + 1 < nchunks:
                plsc.subcore_barrier()

    return segsum


_segsum1 = _sc_segsum_builder(1)
_segsum4 = _sc_segsum_builder(4)


_PPW = NTR // NW
_PB = 32
_PNB = _PPW // _PB
_PIR = _PPW // 128

_head_mesh = plsc.VectorSubcoreMesh(core_axis_name="c", subcore_axis_name="s")


@functools.partial(
    pl.kernel,
    out_type=jax.ShapeDtypeStruct((NTR, H), jnp.float32),
    mesh=_head_mesh,
    scratch_types=[
        pltpu.VMEM((_PIR, 128), jnp.int32),
        pltpu.VMEM((_PIR, 128), jnp.int32),
        pltpu.VMEM((_PIR, 128), jnp.int32),
    ] + [pltpu.VMEM((_PB, H), jnp.float32) for _ in range(6)]
    + [pltpu.SemaphoreType.DMA for _ in range(7)],
)
def _sc_head(h3_hbm, src_hbm, dst_hbm, teid_hbm, out_hbm, *rest):
    teidv, avall, bvall = rest[0:3]
    arows = rest[3:5]
    brows = rest[5:7]
    prod = rest[7:9]
    ga = rest[9:11]
    gb = rest[11:13]
    wsem = rest[13:15]
    psem = rest[15]
    cid = lax.axis_index("c")
    sid = lax.axis_index("s")
    wid = sid * NC + cid

    pltpu.sync_copy(teid_hbm.at[wid], teidv)
    descs = []
    for j in range(_PIR):
        descs.append(pltpu.async_copy(
            src_hbm.at[teidv.at[j]], avall.at[j], psem))
        descs.append(pltpu.async_copy(
            dst_hbm.at[teidv.at[j]], bvall.at[j], psem))
    for d in descs:
        d.wait()

    def body(i, carry):
        gs = []
        for p in range(2):
            b = i * 2 + p
            aidx = avall.at[b // 4, pl.ds((b % 4) * _PB, _PB)]
            bidx = bvall.at[b // 4, pl.ds((b % 4) * _PB, _PB)]
            gs.append((pltpu.async_copy(h3_hbm.at[aidx], arows[p], ga[p]),
                       pltpu.async_copy(h3_hbm.at[bidx], brows[p], gb[p])))
        ws = []
        for p in range(2):
            b = i * 2 + p
            gs[p][0].wait()
            gs[p][1].wait()

            def prow(r, c2):
                for j in range(H // 16):
                    sl = pl.ds(j * 16, 16)
                    prod[p][r, sl] = arows[p][r, sl] * brows[p][r, sl]
                return c2

            lax.fori_loop(0, _PB, prow, 0)
            ws.append(pltpu.async_copy(
                prod[p], out_hbm.at[pl.ds(wid * _PPW + b * _PB, _PB)],
                wsem[p]))
        for w in ws:
            w.wait()
        return carry

    lax.fori_loop(0, _PNB // 2, body, 0)


_BR = 2000


def _mlp1_body(eps_ref, x_ref, p0_ref, p1_ref, w1_ref, b1_ref, w2_ref,
               b2_ref, s1_ref, t1_ref, out_ref):
    xin = (1.0 + eps_ref[0]) * x_ref[...] + p0_ref[0, 0] + p1_ref[0, 0]
    h = jnp.dot(xin, w1_ref[...], preferred_element_type=jnp.float32)
    h = jnp.maximum(h + b1_ref[...], 0.0)
    h = jnp.dot(h, w2_ref[...], preferred_element_type=jnp.float32)
    h = jnp.maximum(h + b2_ref[...], 0.0)
    h = h * s1_ref[...] + t1_ref[...]
    for c in range(4):
        out_ref[c] = h[:, c * DIN:(c + 1) * DIN]


def _tc_mlp1(x, parts, eps1, W1, b1, W2, b2, s1, t1):
    grid = (N // _BR,)
    return pl.pallas_call(
        _mlp1_body,
        grid=grid,
        in_specs=[
            pl.BlockSpec(memory_space=pltpu.SMEM),
            pl.BlockSpec((_BR, DIN), lambda i: (i, 0)),
            pl.BlockSpec((1, 1, _BR, DIN), lambda i: (0, 0, i, 0)),
            pl.BlockSpec((1, 1, _BR, DIN), lambda i: (1, 0, i, 0)),
            pl.BlockSpec((DIN, H), lambda i: (0, 0)),
            pl.BlockSpec((1, H), lambda i: (0, 0)),
            pl.BlockSpec((H, H), lambda i: (0, 0)),
            pl.BlockSpec((1, H), lambda i: (0, 0)),
            pl.BlockSpec((1, H), lambda i: (0, 0)),
            pl.BlockSpec((1, H), lambda i: (0, 0)),
        ],
        out_specs=pl.BlockSpec((4, _BR, DIN), lambda i: (0, i, 0)),
        out_shape=jax.ShapeDtypeStruct((4, N, DIN), jnp.float32),
    )(eps1, x, parts, parts, W1, b1, W2, b2, s1, t1)


def _mlp2_body(eps_ref, h4_ref, q0_ref, q1_ref, w3_ref, b3_ref, s2_ref,
               t2_ref, wl_ref, bl_ref, out_ref):
    h = jnp.concatenate([h4_ref[c] for c in range(4)], axis=1)
    agg = jnp.concatenate(
        [q0_ref[0, c] + q1_ref[0, c] for c in range(4)], axis=1)
    hin = (1.0 + eps_ref[0]) * h + agg
    h2 = jnp.dot(hin, w3_ref[...], preferred_element_type=jnp.float32)
    h2 = jnp.maximum(h2 + b3_ref[...], 0.0)
    h2 = h2 * s2_ref[...] + t2_ref[...]
    h3 = jnp.dot(h2, wl_ref[...], preferred_element_type=jnp.float32)
    out_ref[...] = jnp.maximum(h3 + bl_ref[...], 0.0)


def _tc_mlp2(h4, parts2, eps2, W3, b3, s2, t2, Wl, bl):
    grid = (N // _BR,)
    return pl.pallas_call(
        _mlp2_body,
        grid=grid,
        in_specs=[
            pl.BlockSpec(memory_space=pltpu.SMEM),
            pl.BlockSpec((4, _BR, DIN), lambda i: (0, i, 0)),
            pl.BlockSpec((1, 4, _BR, DIN), lambda i: (0, 0, i, 0)),
            pl.BlockSpec((1, 4, _BR, DIN), lambda i: (1, 0, i, 0)),
            pl.BlockSpec((H, H), lambda i: (0, 0)),
            pl.BlockSpec((1, H), lambda i: (0, 0)),
            pl.BlockSpec((1, H), lambda i: (0, 0)),
            pl.BlockSpec((1, H), lambda i: (0, 0)),
            pl.BlockSpec((H, H), lambda i: (0, 0)),
            pl.BlockSpec((1, H), lambda i: (0, 0)),
        ],
        out_specs=pl.BlockSpec((_BR, H), lambda i: (i, 0)),
        out_shape=jax.ShapeDtypeStruct((N, H), jnp.float32),
    )(eps2, h4, parts2, parts2, W3, b3, s2, t2, Wl, bl)


_BH = 4096


def _headmm_body(p_ref, wf_ref, bf_ref, out_ref):
    o = jnp.dot(p_ref[...], wf_ref[...], preferred_element_type=jnp.float32)
    out_ref[...] = o + bf_ref[...]


def _tc_headmm(p, Wf, bf):
    grid = (NTR // _BH,)
    return pl.pallas_call(
        _headmm_body,
        grid=grid,
        in_specs=[
            pl.BlockSpec((_BH, H), lambda i: (i, 0)),
            pl.BlockSpec((H, C), lambda i: (0, 0)),
            pl.BlockSpec((1, C), lambda i: (0, 0)),
        ],
        out_specs=pl.BlockSpec((_BH, C), lambda i: (i, 0)),
        out_shape=jax.ShapeDtypeStruct((NTR, C), jnp.float32),
    )(p, Wf, bf)


_BN_RS = float(1.0 / (1.0 + 1e-5) ** 0.5)


def kernel(x, edge_index, train_edge_id, eps1, W1, b1, W2, b2, g1, bb1,
           eps2, W3, b3, g2, bb2, Wl, bl, Wf, bf):
    src = edge_index[0]
    dst = edge_index[1]
    eps1s = jnp.reshape(eps1, (1,))
    eps2s = jnp.reshape(eps2, (1,))
    s1 = jnp.reshape(g1 * _BN_RS, (1, H))
    t1 = jnp.reshape(bb1, (1, H))
    s2 = jnp.reshape(g2 * _BN_RS, (1, H))
    t2 = jnp.reshape(bb2, (1, H))
    b1r = jnp.reshape(b1, (1, H))
    b2r = jnp.reshape(b2, (1, H))
    b3r = jnp.reshape(b3, (1, H))
    blr = jnp.reshape(bl, (1, H))
    bfr = jnp.reshape(bf, (1, C))

    pad = EPAD - E
    src_pad = jnp.arange(pad, dtype=jnp.int32) % N
    src3 = jnp.reshape(jnp.concatenate([src, src_pad]),
                       (NW, _NIT, _K, _EB))
    dst_pad = N + jnp.arange(pad, dtype=jnp.int32) % (NP - N)
    dst3 = jnp.reshape(jnp.concatenate([dst, dst_pad]),
                       (NW, _NIT, _K, _EB))
    parts = _segsum1(jnp.reshape(x, (1, N, DIN)), src3, dst3)
    h4 = _tc_mlp1(x, parts, eps1s, W1, b1r, W2, b2r, s1, t1)
    parts2 = _segsum4(h4, src3, dst3)
    h3 = _tc_mlp2(h4, parts2, eps2s, W3, b3r, s2, t2, Wl, blr)
    teid3 = jnp.reshape(train_edge_id, (NW, _PIR, 128))
    p = _sc_head(h3, src, dst, teid3)
    return _tc_headmm(p, Wf, bfr)


# speedup vs baseline: 1.6236x; 1.0047x over previous
import functools

import jax
import jax.numpy as jnp
from jax import lax
from jax.experimental import pallas as pl
from jax.experimental.pallas import tpu as pltpu
from jax.experimental.pallas import tpu_sc as plsc

N = 10000
E = 320000
DIN = 128
H = 512
C = 7
NTR = 65536

NC = 2
NS = 16
NW = NC * NS


_EB = 80
_K = 4
_NIT = 32
_EPW = _NIT * _K * _EB
EPAD = NW * _EPW
NP = 10240
_RPT = NP // NS
_ZR = 32


def _sc_segsum_builder(nchunks):
    mesh = plsc.VectorSubcoreMesh(core_axis_name="c", subcore_axis_name="s")

    @functools.partial(
        pl.kernel,
        out_type=jax.ShapeDtypeStruct((NC, nchunks, NP, DIN), jnp.float32),
        mesh=mesh,
        scratch_types=[
            pltpu.VMEM((_K, _EB), jnp.int32),
            pltpu.VMEM((_K, _EB), jnp.int32),
            pltpu.VMEM((_K, _EB), jnp.int32),
            pltpu.VMEM((_K, _EB), jnp.int32),
        ] + [pltpu.VMEM((_EB, DIN), jnp.float32) for _ in range(_K)] + [
            pltpu.VMEM((_ZR, DIN), jnp.float32),
            pltpu.VMEM_SHARED((NP, DIN), jnp.float32),
        ] + [pltpu.SemaphoreType.DMA for _ in range(2 * _K + 2)],
    )
    def segsum(tab4, src_hbm, dst_hbm, out_hbm, *rest):
        srcv = rest[0:2]
        dstv = rest[2:4]
        rows = rest[4:4 + _K]
        zbuf = rest[4 + _K]
        acc = rest[5 + _K]
        gsem = rest[6 + _K:6 + 2 * _K]
        ssem = rest[6 + 2 * _K:6 + 3 * _K]
        isem = rest[6 + 3 * _K:]
        cid = lax.axis_index("c")
        sid = lax.axis_index("s")
        wid = sid * NC + cid

        def idx_wait(p):
            pltpu.make_async_copy(src_hbm.at[wid, 0], srcv[p], isem[p]).wait()
            pltpu.make_async_copy(dst_hbm.at[wid, 0], dstv[p], isem[p]).wait()

        def zrow(i, carry):
            z = jnp.zeros((16,), jnp.float32)
            for j in range(DIN // 16):
                zbuf[i, pl.ds(j * 16, 16)] = z
            return carry

        lax.fori_loop(0, _ZR, zrow, 0)

        for ci in range(nchunks):
            zds = []
            for i in range(_RPT // _ZR):
                zds.append(pltpu.async_copy(
                    zbuf, acc.at[pl.ds(sid * _RPT + i * _ZR, _ZR)],
                    isem[0]))
            for d in zds:
                d.wait()
            plsc.subcore_barrier()

            tab = tab4.at[ci]

            for p in range(2):
                pltpu.async_copy(src_hbm.at[wid, p], srcv[p], isem[p])
                pltpu.async_copy(dst_hbm.at[wid, p], dstv[p], isem[p])

            def body(i, carry):
                for p in range(2):
                    it = i * 2 + p
                    idx_wait(p)
                    gs = []
                    for k in range(_K):
                        gs.append(pltpu.async_copy(
                            tab.at[srcv[p].at[k]], rows[k], gsem[k]))
                    ss = []
                    for k in range(_K):
                        gs[k].wait()
                        ss.append(pltpu.async_copy(
                            rows[k], acc.at[dstv[p].at[k]], ssem[k],
                            add=True))
                    nxt = jnp.minimum(it + 2, _NIT - 1)
                    pltpu.async_copy(src_hbm.at[wid, nxt], srcv[p], isem[p])
                    for k in range(_K):
                        ss[k].wait()
                    pltpu.async_copy(dst_hbm.at[wid, nxt], dstv[p], isem[p])
                return carry

            lax.fori_loop(0, _NIT // 2, body, 0)
            idx_wait(0)
            idx_wait(1)
            plsc.subcore_barrier()
            pltpu.sync_copy(
                acc.at[pl.ds(sid * _RPT, _RPT)],
                out_hbm.at[cid, ci, pl.ds(sid * _RPT, _RPT)],
            )
            if ci + 1 < nchunks:
                plsc.subcore_barrier()

    return segsum


_segsum1 = _sc_segsum_builder(1)
_segsum4 = _sc_segsum_builder(4)


_PPW = NTR // NW
_PB = 32
_PNB = _PPW // _PB
_PIR = _PPW // 128

_head_mesh = plsc.VectorSubcoreMesh(core_axis_name="c", subcore_axis_name="s")


@functools.partial(
    pl.kernel,
    out_type=jax.ShapeDtypeStruct((NTR, H), jnp.float32),
    mesh=_head_mesh,
    scratch_types=[
        pltpu.VMEM((_PIR, 128), jnp.int32),
        pltpu.VMEM((_PIR, 128), jnp.int32),
        pltpu.VMEM((_PIR, 128), jnp.int32),
    ] + [pltpu.VMEM((_PB, H), jnp.float32) for _ in range(6)]
    + [pltpu.SemaphoreType.DMA for _ in range(7)],
)
def _sc_head(h3_hbm, src_hbm, dst_hbm, teid_hbm, out_hbm, *rest):
    teidv, avall, bvall = rest[0:3]
    arows = rest[3:5]
    brows = rest[5:7]
    prod = rest[7:9]
    ga = rest[9:11]
    gb = rest[11:13]
    wsem = rest[13:15]
    psem = rest[15]
    cid = lax.axis_index("c")
    sid = lax.axis_index("s")
    wid = sid * NC + cid

    pltpu.sync_copy(teid_hbm.at[wid], teidv)
    descs = []
    for j in range(_PIR):
        descs.append(pltpu.async_copy(
            src_hbm.at[teidv.at[j]], avall.at[j], psem))
        descs.append(pltpu.async_copy(
            dst_hbm.at[teidv.at[j]], bvall.at[j], psem))
    for d in descs:
        d.wait()

    def body(i, carry):
        gs = []
        for p in range(2):
            b = i * 2 + p
            aidx = avall.at[b // 4, pl.ds((b % 4) * _PB, _PB)]
            bidx = bvall.at[b // 4, pl.ds((b % 4) * _PB, _PB)]
            gs.append((pltpu.async_copy(h3_hbm.at[aidx], arows[p], ga[p]),
                       pltpu.async_copy(h3_hbm.at[bidx], brows[p], gb[p])))
        ws = []
        for p in range(2):
            b = i * 2 + p
            gs[p][0].wait()
            gs[p][1].wait()

            def prow(r, c2):
                for j in range(H // 16):
                    sl = pl.ds(j * 16, 16)
                    prod[p][r, sl] = arows[p][r, sl] * brows[p][r, sl]
                return c2

            lax.fori_loop(0, _PB, prow, 0)
            ws.append(pltpu.async_copy(
                prod[p], out_hbm.at[pl.ds(wid * _PPW + b * _PB, _PB)],
                wsem[p]))
        for w in ws:
            w.wait()
        return carry

    lax.fori_loop(0, _PNB // 2, body, 0)


_BR = 2000


def _mlp1_body(eps_ref, x_ref, p0_ref, p1_ref, w1_ref, b1_ref, w2_ref,
               b2_ref, s1_ref, t1_ref, out_ref):
    xin = (1.0 + eps_ref[0]) * x_ref[...] + p0_ref[0, 0] + p1_ref[0, 0]
    h = jnp.dot(xin, w1_ref[...], preferred_element_type=jnp.float32)
    h = jnp.maximum(h + b1_ref[...], 0.0)
    h = jnp.dot(h, w2_ref[...], preferred_element_type=jnp.float32)
    h = jnp.maximum(h + b2_ref[...], 0.0)
    h = h * s1_ref[...] + t1_ref[...]
    for c in range(4):
        out_ref[c] = h[:, c * DIN:(c + 1) * DIN]


def _tc_mlp1(x, parts, eps1, W1, b1, W2, b2, s1, t1):
    grid = (N // _BR,)
    return pl.pallas_call(
        _mlp1_body,
        grid=grid,
        in_specs=[
            pl.BlockSpec(memory_space=pltpu.SMEM),
            pl.BlockSpec((_BR, DIN), lambda i: (i, 0)),
            pl.BlockSpec((1, 1, _BR, DIN), lambda i: (0, 0, i, 0)),
            pl.BlockSpec((1, 1, _BR, DIN), lambda i: (1, 0, i, 0)),
            pl.BlockSpec((DIN, H), lambda i: (0, 0)),
            pl.BlockSpec((1, H), lambda i: (0, 0)),
            pl.BlockSpec((H, H), lambda i: (0, 0)),
            pl.BlockSpec((1, H), lambda i: (0, 0)),
            pl.BlockSpec((1, H), lambda i: (0, 0)),
            pl.BlockSpec((1, H), lambda i: (0, 0)),
        ],
        out_specs=pl.BlockSpec((4, _BR, DIN), lambda i: (0, i, 0)),
        out_shape=jax.ShapeDtypeStruct((4, N, DIN), jnp.float32),
    )(eps1, x, parts, parts, W1, b1, W2, b2, s1, t1)


def _mlp2_body(eps_ref, h4_ref, q0_ref, q1_ref, w3_ref, b3_ref, s2_ref,
               t2_ref, wl_ref, bl_ref, out_ref):
    h = jnp.concatenate([h4_ref[c] for c in range(4)], axis=1)
    agg = jnp.concatenate(
        [q0_ref[0, c] + q1_ref[0, c] for c in range(4)], axis=1)
    hin = (1.0 + eps_ref[0]) * h + agg
    h2 = jnp.dot(hin, w3_ref[...], preferred_element_type=jnp.float32)
    h2 = jnp.maximum(h2 + b3_ref[...], 0.0)
    h2 = h2 * s2_ref[...] + t2_ref[...]
    h3 = jnp.dot(h2, wl_ref[...], preferred_element_type=jnp.float32)
    out_ref[...] = jnp.maximum(h3 + bl_ref[...], 0.0)


def _tc_mlp2(h4, parts2, eps2, W3, b3, s2, t2, Wl, bl):
    grid = (N // _BR,)
    return pl.pallas_call(
        _mlp2_body,
        grid=grid,
        in_specs=[
            pl.BlockSpec(memory_space=pltpu.SMEM),
            pl.BlockSpec((4, _BR, DIN), lambda i: (0, i, 0)),
            pl.BlockSpec((1, 4, _BR, DIN), lambda i: (0, 0, i, 0)),
            pl.BlockSpec((1, 4, _BR, DIN), lambda i: (1, 0, i, 0)),
            pl.BlockSpec((H, H), lambda i: (0, 0)),
            pl.BlockSpec((1, H), lambda i: (0, 0)),
            pl.BlockSpec((1, H), lambda i: (0, 0)),
            pl.BlockSpec((1, H), lambda i: (0, 0)),
            pl.BlockSpec((H, H), lambda i: (0, 0)),
            pl.BlockSpec((1, H), lambda i: (0, 0)),
        ],
        out_specs=pl.BlockSpec((_BR, H), lambda i: (i, 0)),
        out_shape=jax.ShapeDtypeStruct((N, H), jnp.float32),
    )(eps2, h4, parts2, parts2, W3, b3, s2, t2, Wl, bl)


_BH = 4096


def _headmm_body(p_ref, wf_ref, bf_ref, out_ref):
    o = jnp.dot(p_ref[...], wf_ref[...], preferred_element_type=jnp.float32)
    out_ref[...] = o + bf_ref[...]


def _tc_headmm(p, Wf, bf):
    grid = (NTR // _BH,)
    return pl.pallas_call(
        _headmm_body,
        grid=grid,
        in_specs=[
            pl.BlockSpec((_BH, H), lambda i: (i, 0)),
            pl.BlockSpec((H, C), lambda i: (0, 0)),
            pl.BlockSpec((1, C), lambda i: (0, 0)),
        ],
        out_specs=pl.BlockSpec((_BH, C), lambda i: (i, 0)),
        out_shape=jax.ShapeDtypeStruct((NTR, C), jnp.float32),
    )(p, Wf, bf)


_BN_RS = float(1.0 / (1.0 + 1e-5) ** 0.5)


def kernel(x, edge_index, train_edge_id, eps1, W1, b1, W2, b2, g1, bb1,
           eps2, W3, b3, g2, bb2, Wl, bl, Wf, bf):
    src = edge_index[0]
    dst = edge_index[1]
    eps1s = jnp.reshape(eps1, (1,))
    eps2s = jnp.reshape(eps2, (1,))
    s1 = jnp.reshape(g1 * _BN_RS, (1, H))
    t1 = jnp.reshape(bb1, (1, H))
    s2 = jnp.reshape(g2 * _BN_RS, (1, H))
    t2 = jnp.reshape(bb2, (1, H))
    b1r = jnp.reshape(b1, (1, H))
    b2r = jnp.reshape(b2, (1, H))
    b3r = jnp.reshape(b3, (1, H))
    blr = jnp.reshape(bl, (1, H))
    bfr = jnp.reshape(bf, (1, C))

    pad = EPAD - E
    src_pad = jnp.arange(pad, dtype=jnp.int32) % N
    src3 = jnp.reshape(jnp.concatenate([src, src_pad]),
                       (NW, _NIT, _K, _EB))
    dst_pad = N + jnp.arange(pad, dtype=jnp.int32) % (NP - N)
    dst3 = jnp.reshape(jnp.concatenate([dst, dst_pad]),
                       (NW, _NIT, _K, _EB))
    parts = _segsum1(jnp.reshape(x, (1, N, DIN)), src3, dst3)
    h4 = _tc_mlp1(x, parts, eps1s, W1, b1r, W2, b2r, s1, t1)
    parts2 = _segsum4(h4, src3, dst3)
    h3 = _tc_mlp2(h4, parts2, eps2s, W3, b3r, s2, t2, Wl, blr)
    teid3 = jnp.reshape(train_edge_id, (NW, _PIR, 128))
    p = _sc_head(h3, src, dst, teid3)
    return _tc_headmm(p, Wf, bfr)
